# baseline (speedup 1.0000x reference)
import functools

import jax
import jax.numpy as jnp
from jax import lax
from jax.experimental import pallas as pl
from jax.experimental.pallas import tpu as pltpu
from jax.experimental.pallas import tpu_sc as plsc

_B, _N, _M, _D = 4, 4096, 16384, 3
_KX, _KS, _P, _H, _O = 256, 128, 128, 256, 128
_K = 16
_MB = 256
_W = 1536
_INF = jnp.inf

_NC, _NS, _L = 2, 16, 16
_CQ = 16


def _make_select_kernel(n):

  def body(start_ref, narrow_ref,
           posT_ref, bx_ref, ps_ref, bs_ref,
           idx_ref, wn_ref, cur_ref):
    i = pl.program_id(0)
    ps = ps_ref[...]
    py2 = jnp.sum(ps * ps, axis=1, keepdims=True)

    def run_path(width, s):
      if s is None:
        csl = slice(None)
        col0 = 0
      else:
        csl = pl.ds(s, width)
        col0 = s
      posTw = posT_ref[:, csl]
      px2 = jnp.sum(posTw * posTw, axis=0, keepdims=True)
      d2 = py2 + px2 - 2.0 * jnp.dot(ps, posTw,
                                     preferred_element_type=jnp.float32)
      d2 = jnp.where(bs_ref[...] != bx_ref[:, csl], _INF, d2)
      cols = jax.lax.broadcasted_iota(jnp.int32, (_MB, width), 1) + col0
      keys = (jax.lax.bitcast_convert_type(d2, jnp.int32)
              & (-4096)) | cols
      cur_ref[:, :width] = keys

      ms = []
      for _ in range(_K):
        c = cur_ref[:, :width]
        m = jnp.min(c, axis=1, keepdims=True)
        ms.append(m)
        cur_ref[:, :width] = jnp.where(c == m, 0x7FFFFFFF, c)
      k16 = jnp.concatenate(ms, axis=1)
      idx_ref[...] = k16 & 0xFFF
      d2k = jax.lax.bitcast_convert_type(k16 & (-4096), jnp.float32)
      w = 1.0 / jnp.maximum(d2k, 1e-16)
      wn = w / jnp.sum(w, axis=1, keepdims=True)
      erow = jax.lax.broadcasted_iota(jnp.int32, (_K, _K * _L), 0)
      ecol = jax.lax.broadcasted_iota(jnp.int32, (_K, _K * _L), 1)
      expand = jnp.where(erow == ecol // _L, 1.0, 0.0)
      wn_ref[...] = jnp.dot(wn, expand, preferred_element_type=jnp.float32)

    @pl.when(narrow_ref[i] == 1)
    def _():
      run_path(_W, pl.multiple_of(start_ref[i], 128))

    @pl.when(narrow_ref[i] == 0)
    def _():
      run_path(n, None)

  return body


def _select(posT, bx, pos_skip, bs, start, narrow, M, N):
    nblocks = M // _MB
    const = lambda i: (0, 0)
    smem = lambda shape: pl.BlockSpec(shape, lambda i: tuple(0 for _ in shape),
                                      memory_space=pltpu.SMEM)
    return pl.pallas_call(
        _make_select_kernel(N),
        grid=(nblocks,),
        in_specs=[
            smem((nblocks,)), smem((nblocks,)),
            pl.BlockSpec((_D, N), const),
            pl.BlockSpec((1, N), const),
            pl.BlockSpec((_MB, _D), lambda i: (i, 0)),
            pl.BlockSpec((_MB, 1), lambda i: (i, 0)),
        ],
        out_specs=[pl.BlockSpec((_MB, _K), lambda i: (i, 0)),
                   pl.BlockSpec((_MB, _K * _L), lambda i: (i, 0))],
        out_shape=[jax.ShapeDtypeStruct((M, _K), jnp.int32),
                   jax.ShapeDtypeStruct((M, _K * _L), jnp.float32)],
        scratch_shapes=[pltpu.VMEM((_MB, N), jnp.int32)],
    )(start, narrow, posT, bx, pos_skip, bs)


def _sc_gather(x, idx, wn):
    M = idx.shape[0]
    q_per_w = M // (_NC * _NS)
    nchunks = q_per_w // _CQ
    mesh = plsc.VectorSubcoreMesh(core_axis_name="c", subcore_axis_name="s")

    @functools.partial(
        pl.kernel, mesh=mesh,
        out_type=jax.ShapeDtypeStruct((M, _KX), jnp.float32),
        scratch_types=[
            pltpu.VMEM((_CQ, _K), jnp.int32),
            pltpu.VMEM((_CQ, _K * _L), jnp.float32),
            pltpu.VMEM((_CQ * _K,), jnp.int32),
            pltpu.VMEM((_CQ * _K, _KX), jnp.float32),
            pltpu.VMEM((_CQ, _KX), jnp.float32),
            pltpu.SemaphoreType.DMA,
        ],
    )
    def k(x_hbm, idx_hbm, wn_hbm, out_hbm,
          idxc_v, wnc_v, flat_v, rows_v, xi_v, sem):
        wid = lax.axis_index("s") * _NC + lax.axis_index("c")
        base = wid * q_per_w

        def chunk(c, carry):
            q0 = base + c * _CQ
            pltpu.sync_copy(idx_hbm.at[pl.ds(q0, _CQ), :], idxc_v)
            pltpu.sync_copy(wn_hbm.at[pl.ds(q0, _CQ), :], wnc_v)

            def flatten(q, carry2):
                flat_v[pl.ds(q * _K, _K)] = idxc_v[q, :]
                return carry2

            lax.fori_loop(0, _CQ, flatten, 0)
            pltpu.async_copy(x_hbm.at[flat_v], rows_v, sem).wait()

            def pool(q, carry2):
                wj = [wnc_v[q, pl.ds(j * _L, _L)] for j in range(_K)]
                for f in range(_KX // _L):
                    fs = pl.ds(f * _L, _L)
                    acc = jnp.zeros((_L,), jnp.float32)
                    for j in range(_K):
                        acc = acc + wj[j] * rows_v[q * _K + j, fs]
                    xi_v[q, fs] = acc
                return carry2

            lax.fori_loop(0, _CQ, pool, 0)
            pltpu.sync_copy(xi_v, out_hbm.at[pl.ds(q0, _CQ), :])
            return carry

        lax.fori_loop(0, nchunks, chunk, 0)

    return k(x, idx, wn)


def _make_mlp_kernel(blocks_per_par):

  def body(par_ref, xi_ref, xs_ref,
           W1_ref, b1_ref, W2_ref, b2_ref, Wp_ref, bp_ref, out_ref):
    xc = jnp.concatenate([xi_ref[...], xs_ref[...]], axis=1)
    h = jnp.tanh(jnp.dot(xc, W1_ref[...], preferred_element_type=jnp.float32)
                 + b1_ref[...])
    h = jnp.dot(h, W2_ref[...], preferred_element_type=jnp.float32) + b2_ref[...]
    g_all = jnp.tanh(jnp.dot(par_ref[...], Wp_ref[...],
                             preferred_element_type=jnp.float32)
                     + bp_ref[...])
    pid = pl.program_id(0) // blocks_per_par
    rows = jax.lax.broadcasted_iota(jnp.int32, g_all.shape, 0)
    g = jnp.sum(jnp.where(rows == pid, g_all, 0.0), axis=0, keepdims=True)
    out_ref[...] = h * g

  return body


def _mlp(par_rows, xi, x_skip, W1, b1, W2, b2, Wp, bp, n_repeats):
    M = xi.shape[0]
    nblocks = M // _MB
    const = lambda i: (0, 0)
    return pl.pallas_call(
        _make_mlp_kernel(n_repeats // _MB),
        grid=(nblocks,),
        in_specs=[
            pl.BlockSpec((par_rows.shape[0], _P), const),
            pl.BlockSpec((_MB, _KX), lambda i: (i, 0)),
            pl.BlockSpec((_MB, _KS), lambda i: (i, 0)),
            pl.BlockSpec((_KX + _KS, _H), const),
            pl.BlockSpec((1, _H), const),
            pl.BlockSpec((_H, _O), const),
            pl.BlockSpec((1, _O), const),
            pl.BlockSpec((_P, _O), const),
            pl.BlockSpec((1, _O), const),
        ],
        out_specs=pl.BlockSpec((_MB, _O), lambda i: (i, 0)),
        out_shape=jax.ShapeDtypeStruct((M, _O), jnp.float32),
    )(par_rows, xi, x_skip,
      W1, b1.reshape(1, _H), W2, b2.reshape(1, _O), Wp, bp.reshape(1, _O))


def kernel(par_embedding, x, pos, batch, x_skip, pos_skip, batch_skip,
           W1, b1, W2, b2, Wp, bp):
    M, N = pos_skip.shape[0], pos.shape[0]
    n_repeats = M // par_embedding.shape[0]
    par_rows = par_embedding.reshape(par_embedding.shape[0], par_embedding.shape[-1])
    posT = pos.T
    batch = batch.astype(jnp.int32)
    batch_skip = batch_skip.astype(jnp.int32)
    bx = batch.astype(jnp.float32).reshape(1, N)
    bs = batch_skip.astype(jnp.float32).reshape(M, 1)

    nblocks = M // _MB
    blk_lo = batch_skip[:: _MB]
    blk_hi = batch_skip[_MB - 1:: _MB]
    col_lo = jnp.searchsorted(batch, blk_lo, side="left").astype(jnp.int32)
    col_hi = (jnp.searchsorted(batch, blk_hi, side="right") - 1).astype(jnp.int32)
    a = (col_lo // 128) * 128
    narrow = ((col_hi - a + 1) <= _W).astype(jnp.int32)
    start = jnp.minimum(a, N - _W).astype(jnp.int32)

    idx, wn = _select(posT, bx, pos_skip, bs, start, narrow, M, N)
    xi = _sc_gather(x, idx, wn)
    out = _mlp(par_rows, xi, x_skip, W1, b1, W2, b2, Wp, bp, n_repeats)
    return (out, pos_skip, batch_skip)


# speedup vs baseline: 2.7465x; 2.7465x over previous
import jax
import jax.numpy as jnp
from jax.experimental import pallas as pl
from jax.experimental.pallas import tpu as pltpu

_B, _N, _M, _D = 4, 4096, 16384, 3
_KX, _KS, _P, _H, _O = 256, 128, 128, 256, 128
_K = 16
_MB = 256
_W = 1280
_INF = jnp.inf


def _make_kernel(n, blocks_per_par):

  def body(start_ref, narrow_ref,
           par_ref, posT_ref, bx_ref, x_ref,
           ps_ref, bs_ref, xs_ref,
           W1_ref, b1_ref, W2_ref, b2_ref, Wp_ref, bp_ref,
           out_ref,
           keys_ref, yacc_ref, wacc_ref):
    i = pl.program_id(0)
    ps = ps_ref[...]
    py2 = jnp.sum(ps * ps, axis=1, keepdims=True)

    def run_path(width, s):
      if s is None:
        csl = slice(None)
        rsl = slice(None)
      else:
        csl = pl.ds(s, width)
        rsl = pl.ds(s, width)
      posTw = posT_ref[:, csl]
      px2 = jnp.sum(posTw * posTw, axis=0, keepdims=True)
      d2 = py2 + px2 - 2.0 * jnp.dot(ps, posTw,
                                     preferred_element_type=jnp.float32)
      d2 = jnp.where(bs_ref[...] != bx_ref[:, csl], _INF, d2)
      keys_ref[:, :width] = d2

      t = jnp.full((_MB, 1), -_INF, jnp.float32)
      for _ in range(_K):
        c = keys_ref[:, :width]
        t = jnp.min(jnp.where(c > t, c, _INF), axis=1, keepdims=True)

      d2v = keys_ref[:, :width]
      w = jnp.where(d2v <= t, 1.0 / jnp.maximum(d2v, 1e-16), 0.0)
      wacc_ref[...] = jnp.sum(w, axis=1, keepdims=True)
      yacc_ref[...] = jnp.dot(w, x_ref[rsl, :],
                              preferred_element_type=jnp.float32)

    @pl.when(narrow_ref[i] == 1)
    def _():
      s = pl.multiple_of(start_ref[i], 128)
      run_path(_W, s)

    @pl.when(narrow_ref[i] == 0)
    def _():
      run_path(n, None)

    y = yacc_ref[...] / wacc_ref[...]
    xc = jnp.concatenate([y, xs_ref[...]], axis=1)
    h = jnp.tanh(jnp.dot(xc, W1_ref[...], preferred_element_type=jnp.float32)
                 + b1_ref[...])
    h = jnp.dot(h, W2_ref[...], preferred_element_type=jnp.float32) + b2_ref[...]
    g_all = jnp.tanh(jnp.dot(par_ref[...], Wp_ref[...],
                             preferred_element_type=jnp.float32)
                     + bp_ref[...])
    pid = pl.program_id(0) // blocks_per_par
    rows = jax.lax.broadcasted_iota(jnp.int32, g_all.shape, 0)
    g = jnp.sum(jnp.where(rows == pid, g_all, 0.0), axis=0, keepdims=True)
    out_ref[...] = h * g

  return body


def kernel(par_embedding, x, pos, batch, x_skip, pos_skip, batch_skip,
           W1, b1, W2, b2, Wp, bp):
    M, N = pos_skip.shape[0], pos.shape[0]
    n_repeats = M // par_embedding.shape[0]
    par_rows = par_embedding.reshape(par_embedding.shape[0], par_embedding.shape[-1])
    posT = pos.T
    batch = batch.astype(jnp.int32)
    batch_skip = batch_skip.astype(jnp.int32)
    bx = batch.astype(jnp.float32).reshape(1, N)
    bs = batch_skip.astype(jnp.float32).reshape(M, 1)

    nblocks = M // _MB
    blk_lo = batch_skip[:: _MB]
    blk_hi = batch_skip[_MB - 1:: _MB]
    col_lo = jnp.searchsorted(batch, blk_lo, side="left").astype(jnp.int32)
    col_hi = (jnp.searchsorted(batch, blk_hi, side="right") - 1).astype(jnp.int32)
    a = (col_lo // 128) * 128
    narrow = ((col_hi - a + 1) <= _W).astype(jnp.int32)
    start = jnp.minimum(a, N - _W).astype(jnp.int32)

    grid = (nblocks,)
    const = lambda i: (0, 0)
    smem = lambda shape: pl.BlockSpec(shape, lambda i: tuple(0 for _ in shape),
                                      memory_space=pltpu.SMEM)
    out = pl.pallas_call(
        _make_kernel(N, n_repeats // _MB),
        grid=grid,
        in_specs=[
            smem((nblocks,)), smem((nblocks,)),
            pl.BlockSpec((par_rows.shape[0], _P), const),
            pl.BlockSpec((_D, N), const),
            pl.BlockSpec((1, N), const),
            pl.BlockSpec((N, _KX), const),
            pl.BlockSpec((_MB, _D), lambda i: (i, 0)),
            pl.BlockSpec((_MB, 1), lambda i: (i, 0)),
            pl.BlockSpec((_MB, _KS), lambda i: (i, 0)),
            pl.BlockSpec((_KX + _KS, _H), const),
            pl.BlockSpec((1, _H), const),
            pl.BlockSpec((_H, _O), const),
            pl.BlockSpec((1, _O), const),
            pl.BlockSpec((_P, _O), const),
            pl.BlockSpec((1, _O), const),
        ],
        out_specs=pl.BlockSpec((_MB, _O), lambda i: (i, 0)),
        out_shape=jax.ShapeDtypeStruct((M, _O), jnp.float32),
        scratch_shapes=[
            pltpu.VMEM((_MB, N), jnp.float32),
            pltpu.VMEM((_MB, _KX), jnp.float32),
            pltpu.VMEM((_MB, 1), jnp.float32),
        ],
    )(start, narrow,
      par_rows, posT, bx, x,
      pos_skip, bs, x_skip,
      W1, b1.reshape(1, _H), W2, b2.reshape(1, _O), Wp, bp.reshape(1, _O))
    return (out, pos_skip, batch_skip)
